# baseline (device time: 113567 ns/iter reference)
import jax
import jax.numpy as jnp
from jax import lax
from jax.experimental import pallas as pl
from jax.experimental.pallas import tpu as pltpu

N_DEV = 4
KC = 512
Q = 4
N_HOP = N_DEV - 1


def kernel(x, w_mat, scale_x, scale_w):
    m_per, k = x.shape
    n = w_mat.shape[1]
    n_per = n // N_DEV
    m_half = m_per // 2
    seg = m_half // Q
    n_kc = k // KC

    def body(x_hbm, w_hbm, sx_ref, sw_ref, out_hbm,
             comm_r, comm_l, x_stage, w_stage, w8, acc,
             send_r, recv_r, send_l, recv_l, load_sems, out_sems):
        my_pos = lax.axis_index("i")
        left = lax.rem(my_pos + (N_DEV - 1), N_DEV)
        right = lax.rem(my_pos + 1, N_DEV)
        col0 = my_pos * n_per

        x_cp = pltpu.make_async_copy(x_hbm, x_stage, load_sems.at[0])
        x_cp.start()

        def w_cp(c, slot):
            return pltpu.make_async_copy(
                w_hbm.at[pl.ds(c * KC, KC), pl.ds(col0, n_per)],
                w_stage.at[slot],
                load_sems.at[1 + slot],
            )

        w_cp(0, 0).start()
        w_cp(1, 1).start()

        x_cp.wait()
        comm_r[0] = x_stage[: m_half, :].astype(jnp.float8_e5m2)
        comm_l[0] = x_stage[m_half:, :].astype(jnp.float8_e5m2)

        barrier_sem = pltpu.get_barrier_semaphore()
        for nbr in (left, right):
            pl.semaphore_signal(
                barrier_sem, inc=1,
                device_id=(nbr,), device_id_type=pl.DeviceIdType.MESH,
            )
        pl.semaphore_wait(barrier_sem, 2)

        scale = sx_ref[0] * sw_ref[0]

        def seg_rdma(d, h, q):
            comm, ss, rs, tgt = (
                (comm_r, send_r, recv_r, right) if d == 0
                else (comm_l, send_l, recv_l, left)
            )
            return pltpu.make_async_remote_copy(
                src_ref=comm.at[h, pl.ds(q * seg, seg)],
                dst_ref=comm.at[h + 1, pl.ds(q * seg, seg)],
                send_sem=ss.at[h, q],
                recv_sem=rs.at[h, q],
                device_id=(tgt,),
                device_id_type=pl.DeviceIdType.MESH,
            )

        for q in range(Q):
            seg_rdma(0, 0, q).start()
            seg_rdma(1, 0, q).start()

        for c in range(n_kc):
            slot = c % 2
            w_cp(c, slot).wait()
            w8[pl.ds(c * KC, KC), :] = w_stage[slot].astype(jnp.float8_e5m2)
            if c + 2 < n_kc:
                w_cp(c + 2, slot).start()

        out_copies = []

        def gemm_store(s, top):
            b = len(out_copies)
            slot = b % 2
            if b >= 2:
                out_copies[b - 2].wait()
            origin = lax.rem(my_pos + (N_DEV - s if top else s), N_DEV)
            chunk = comm_r[s] if top else comm_l[s]
            a = jnp.dot(chunk, w8[...], preferred_element_type=jnp.float32)
            acc[slot] = jnp.maximum(a * scale, 0.0)
            row0 = origin * m_per + (0 if top else m_half)
            cp = pltpu.make_async_copy(
                acc.at[slot],
                out_hbm.at[pl.ds(row0, m_half), :],
                out_sems.at[slot],
            )
            cp.start()
            out_copies.append(cp)

        for q in range(Q):
            for d in (0, 1):
                seg_rdma(d, 0, q).wait_recv()
                seg_rdma(d, 1, q).start()
        gemm_store(0, True)
        gemm_store(0, False)
        for q in range(Q):
            for d in (0, 1):
                seg_rdma(d, 1, q).wait_recv()
                seg_rdma(d, 2, q).start()
        gemm_store(1, True)
        gemm_store(1, False)
        gemm_store(2, True)
        gemm_store(2, False)
        for q in range(Q):
            seg_rdma(0, 2, q).wait_recv()
            seg_rdma(1, 2, q).wait_recv()
        gemm_store(3, True)
        gemm_store(3, False)

        for h in range(N_HOP):
            for q in range(Q):
                seg_rdma(0, h, q).wait_send()
                seg_rdma(1, h, q).wait_send()
        out_copies[-2].wait()
        out_copies[-1].wait()

    out_shape = jax.ShapeDtypeStruct((N_DEV * m_per, n_per), jnp.float32)
    return pl.pallas_call(
        body,
        out_shape=out_shape,
        in_specs=[
            pl.BlockSpec(memory_space=pl.ANY),
            pl.BlockSpec(memory_space=pl.ANY),
            pl.BlockSpec(memory_space=pltpu.SMEM),
            pl.BlockSpec(memory_space=pltpu.SMEM),
        ],
        out_specs=pl.BlockSpec(memory_space=pl.ANY),
        scratch_shapes=[
            pltpu.VMEM((N_DEV, m_half, k), jnp.float8_e5m2),
            pltpu.VMEM((N_DEV, m_half, k), jnp.float8_e5m2),
            pltpu.VMEM((m_per, k), jnp.float32),
            pltpu.VMEM((2, KC, n_per), jnp.float32),
            pltpu.VMEM((k, n_per), jnp.float8_e5m2),
            pltpu.VMEM((2, m_half, n_per), jnp.float32),
            pltpu.SemaphoreType.DMA((N_HOP, Q)),
            pltpu.SemaphoreType.DMA((N_HOP, Q)),
            pltpu.SemaphoreType.DMA((N_HOP, Q)),
            pltpu.SemaphoreType.DMA((N_HOP, Q)),
            pltpu.SemaphoreType.DMA((3,)),
            pltpu.SemaphoreType.DMA((2,)),
        ],
        compiler_params=pltpu.CompilerParams(
            collective_id=0, vmem_limit_bytes=100 * 1024 * 1024
        ),
    )(x, w_mat, scale_x, scale_w)


# device time: 104716 ns/iter; 1.0845x vs baseline; 1.0845x over previous
import os

import jax
import jax.numpy as jnp
from jax import lax
from jax.experimental import pallas as pl
from jax.experimental.pallas import tpu as pltpu

KMODE = os.environ.get("KMODE", "full")

N_DEV = 4
KC = 512
Q = 4
N_HOP = N_DEV - 1


def kernel(x, w_mat, scale_x, scale_w):
    m_per, k = x.shape
    n = w_mat.shape[1]
    n_per = n // N_DEV
    m_half = m_per // 2
    seg = m_half // Q
    n_kc = k // KC

    def body(x_hbm, w_hbm, sx_ref, sw_ref, out_hbm,
             comm_r, comm_l, x_stage, w_stage, w8, acc,
             send_r, recv_r, send_l, recv_l, xload_sems, wload_sems,
             out_sems):
        my_pos = lax.axis_index("i")
        left = lax.rem(my_pos + (N_DEV - 1), N_DEV)
        right = lax.rem(my_pos + 1, N_DEV)
        col0 = my_pos * n_per

        comm_only = KMODE != "full"
        qe = Q // 2 if KMODE == "commhalf" else Q

        def x_cp(d, q):
            r0 = (0 if d == 0 else m_half) + q * seg
            return pltpu.make_async_copy(
                x_hbm.at[pl.ds(r0, seg), :],
                x_stage.at[pl.ds(r0, seg), :],
                xload_sems.at[d, q],
            )

        for q in range(Q):
            x_cp(0, q).start()
            x_cp(1, q).start()

        def w_cp(c, slot):
            return pltpu.make_async_copy(
                w_hbm.at[pl.ds(c * KC, KC), pl.ds(col0, n_per)],
                w_stage.at[slot],
                wload_sems.at[slot],
            )

        if not comm_only:
            w_cp(0, 0).start()
            w_cp(1, 1).start()

        barrier_sem = pltpu.get_barrier_semaphore()
        for nbr in (left, right):
            pl.semaphore_signal(
                barrier_sem, inc=1,
                device_id=(nbr,), device_id_type=pl.DeviceIdType.MESH,
            )
        pl.semaphore_wait(barrier_sem, 2)

        scale = sx_ref[0] * sw_ref[0]

        def seg_rdma(d, h, q):
            comm, ss, rs, tgt = (
                (comm_r, send_r, recv_r, right) if d == 0
                else (comm_l, send_l, recv_l, left)
            )
            return pltpu.make_async_remote_copy(
                src_ref=comm.at[h, pl.ds(q * seg, seg)],
                dst_ref=comm.at[h + 1, pl.ds(q * seg, seg)],
                send_sem=ss.at[h, q],
                recv_sem=rs.at[h, q],
                device_id=(tgt,),
                device_id_type=pl.DeviceIdType.MESH,
            )

        for q in range(qe):
            for d in (0, 1):
                x_cp(d, q).wait()
                r0 = (0 if d == 0 else m_half) + q * seg
                comm = comm_r if d == 0 else comm_l
                comm[0, pl.ds(q * seg, seg), :] = x_stage[
                    pl.ds(r0, seg), :
                ].astype(jnp.float8_e5m2)
                seg_rdma(d, 0, q).start()
        for q in range(qe, Q):
            x_cp(0, q).wait()
            x_cp(1, q).wait()

        if not comm_only:
            for c in range(n_kc):
                slot = c % 2
                w_cp(c, slot).wait()
                w8[pl.ds(c * KC, KC), :] = w_stage[slot].astype(
                    jnp.float8_e5m2
                )
                if c + 2 < n_kc:
                    w_cp(c + 2, slot).start()

        out_copies = []

        def gemm_rows(s, top, row_off, nrows):
            b = len(out_copies)
            slot = b % 2
            if b >= 2:
                out_copies[b - 2].wait()
            origin = lax.rem(my_pos + (N_DEV - s if top else s), N_DEV)
            chunk = (comm_r if top else comm_l)[s, pl.ds(row_off, nrows), :]
            a = jnp.dot(chunk, w8[...], preferred_element_type=jnp.float32)
            acc[slot, pl.ds(0, nrows)] = jnp.maximum(a * scale, 0.0)
            row0 = origin * m_per + (0 if top else m_half) + row_off
            cp = pltpu.make_async_copy(
                acc.at[slot, pl.ds(0, nrows)],
                out_hbm.at[pl.ds(row0, nrows), :],
                out_sems.at[slot],
            )
            cp.start()
            out_copies.append(cp)

        def gemm_store(s, top):
            gemm_rows(s, top, 0, m_half)

        for q in range(qe):
            for d in (0, 1):
                seg_rdma(d, 0, q).wait_recv()
                seg_rdma(d, 1, q).start()
        if not comm_only:
            gemm_store(0, True)
            gemm_store(0, False)
        for q in range(qe):
            for d in (0, 1):
                seg_rdma(d, 1, q).wait_recv()
                seg_rdma(d, 2, q).start()
        if not comm_only:
            gemm_store(1, True)
            gemm_store(1, False)
            gemm_store(2, True)
            gemm_store(2, False)
        for p in range(max(qe // 2, 1)):
            for q in (2 * p, 2 * p + 1):
                if q < qe:
                    seg_rdma(0, 2, q).wait_recv()
                    seg_rdma(1, 2, q).wait_recv()
            if not comm_only:
                nrows = min(2 * seg, (qe - 2 * p) * seg)
                gemm_rows(3, True, 2 * p * seg, nrows)
                gemm_rows(3, False, 2 * p * seg, nrows)

        for h in range(N_HOP):
            for q in range(qe):
                seg_rdma(0, h, q).wait_send()
                seg_rdma(1, h, q).wait_send()
        if not comm_only:
            out_copies[-2].wait()
            out_copies[-1].wait()

    out_shape = jax.ShapeDtypeStruct((N_DEV * m_per, n_per), jnp.float32)
    return pl.pallas_call(
        body,
        out_shape=out_shape,
        in_specs=[
            pl.BlockSpec(memory_space=pl.ANY),
            pl.BlockSpec(memory_space=pl.ANY),
            pl.BlockSpec(memory_space=pltpu.SMEM),
            pl.BlockSpec(memory_space=pltpu.SMEM),
        ],
        out_specs=pl.BlockSpec(memory_space=pl.ANY),
        scratch_shapes=[
            pltpu.VMEM((N_DEV, m_half, k), jnp.float8_e5m2),
            pltpu.VMEM((N_DEV, m_half, k), jnp.float8_e5m2),
            pltpu.VMEM((m_per, k), jnp.float32),
            pltpu.VMEM((2, KC, n_per), jnp.float32),
            pltpu.VMEM((k, n_per), jnp.float8_e5m2),
            pltpu.VMEM((2, m_half, n_per), jnp.float32),
            pltpu.SemaphoreType.DMA((N_HOP, Q)),
            pltpu.SemaphoreType.DMA((N_HOP, Q)),
            pltpu.SemaphoreType.DMA((N_HOP, Q)),
            pltpu.SemaphoreType.DMA((N_HOP, Q)),
            pltpu.SemaphoreType.DMA((2, Q)),
            pltpu.SemaphoreType.DMA((2,)),
            pltpu.SemaphoreType.DMA((2,)),
        ],
        compiler_params=pltpu.CompilerParams(
            collective_id=0, vmem_limit_bytes=100 * 1024 * 1024
        ),
    )(x, w_mat, scale_x, scale_w)
